# baseline (speedup 1.0000x reference)
import functools
import jax
import jax.numpy as jnp
from jax import lax
from jax.experimental import pallas as pl
from jax.experimental.pallas import tpu as pltpu
from jax.experimental.pallas import tpu_sc as plsc

LAT = 128
_B = 64
_NS = 16

INTERP = False


def _fill_body(xs_ref, ys_ref, out_ref, idx_ref):
    i = pl.program_id(0)
    x = xs_ref[0, 0, :].astype(jnp.int32)
    y = ys_ref[0, 0, :].astype(jnp.int32)
    cfg = i * _B + lax.broadcasted_iota(jnp.int32, (_B,), 0)
    idx_ref[0, 0, :] = cfg * (LAT * LAT) + y * LAT + x
    pos = lax.broadcasted_iota(jnp.int32, (_B, LAT, LAT), 1) * LAT + \
        lax.broadcasted_iota(jnp.int32, (_B, LAT, LAT), 2)
    out_ref[...] = jnp.where(pos == 0, 1.0, 0.0).reshape(_B, 1, LAT, LAT)


def _tc_fill(xs3, ys3, n):
    g = n // _B
    return pl.pallas_call(
        _fill_body,
        grid=(g,),
        in_specs=[
            pl.BlockSpec((1, 1, _B), lambda i: (i, 0, 0)),
            pl.BlockSpec((1, 1, _B), lambda i: (i, 0, 0)),
        ],
        out_specs=[
            pl.BlockSpec((_B, 1, LAT, LAT), lambda i: (i, 0, 0, 0)),
            pl.BlockSpec((1, 1, _B), lambda i: (i, 0, 0)),
        ],
        out_shape=[
            jax.ShapeDtypeStruct((n, 1, LAT, LAT), jnp.float32),
            jax.ShapeDtypeStruct((g, 1, _B), jnp.int32),
        ],
        interpret=INTERP,
    )(xs3, ys3)


def _make_sc_scatter(n):
    bpw = n // _NS
    ndma = bpw // 128
    mesh = plsc.VectorSubcoreMesh(
        core_axis_name="c", subcore_axis_name="s", num_cores=1)

    @functools.partial(
        pl.kernel,
        mesh=mesh,
        scratch_types=[
            [pltpu.VMEM((128,), jnp.int32) for _ in range(ndma)],
            pltpu.VMEM((128,), jnp.float32),
            pltpu.SemaphoreType.DMA,
            pltpu.SemaphoreType.DMA,
        ],
        interpret=INTERP,
    )
    def sc_scatter(idx_hbm, masks_ref, idx_vs, val_v, sem_in, sem_out):
        wid = lax.axis_index("s")
        base = wid * bpw
        loads = [
            pltpu.async_copy(
                idx_hbm.at[pl.ds(base + d * 128, 128)], idx_vs[d], sem_in)
            for d in range(ndma)
        ]
        for j in range(8):
            val_v[pl.ds(j * 16, 16)] = jnp.full((16,), -1.0, jnp.float32)
        for c in loads:
            c.wait()
        stores = [
            pltpu.async_copy(val_v, masks_ref.at[idx_vs[d]], sem_out)
            for d in range(ndma)
        ]
        for c in stores:
            c.wait()

    return sc_scatter


@jax.jit
def _run(x_seps, y_seps):
    n = x_seps.shape[0]
    g = n // _B
    xs3 = x_seps.reshape(g, 1, _B)
    ys3 = y_seps.reshape(g, 1, _B)
    filled, idx3 = _tc_fill(xs3, ys3, n)
    flat = jax.new_ref(filled.reshape(n * LAT * LAT))
    _make_sc_scatter(n)(idx3.reshape(n), flat)
    return jax.freeze(flat).reshape(n, 1, LAT, LAT)


def kernel(x_seps, y_seps):
    return _run(x_seps, y_seps)


# speedup vs baseline: 1.0375x; 1.0375x over previous
import functools
import jax
import jax.numpy as jnp
from jax import lax
from jax.experimental import pallas as pl
from jax.experimental.pallas import tpu as pltpu
from jax.experimental.pallas import tpu_sc as plsc

LAT = 128
_B = 64
_NS = 16

INTERP = False


def _fill_body(out_ref):
    pos = lax.broadcasted_iota(jnp.int32, (_B, LAT, LAT), 1) * LAT + \
        lax.broadcasted_iota(jnp.int32, (_B, LAT, LAT), 2)
    out_ref[...] = jnp.where(pos == 0, 1.0, 0.0).reshape(_B, 1, LAT, LAT)


def _tc_fill(n):
    g = n // _B
    return pl.pallas_call(
        _fill_body,
        grid=(g,),
        out_specs=pl.BlockSpec((_B, 1, LAT, LAT), lambda i: (i, 0, 0, 0)),
        out_shape=jax.ShapeDtypeStruct((n, 1, LAT, LAT), jnp.float32),
        interpret=INTERP,
    )()


def _make_sc_scatter(n):
    bpw = n // _NS
    ndma = bpw // 128
    mesh = plsc.VectorSubcoreMesh(
        core_axis_name="c", subcore_axis_name="s", num_cores=1)

    @functools.partial(
        pl.kernel,
        mesh=mesh,
        scratch_types=[
            pltpu.VMEM((bpw,), jnp.float32),
            pltpu.VMEM((bpw,), jnp.float32),
            [pltpu.VMEM((128,), jnp.int32) for _ in range(ndma)],
            pltpu.VMEM((128,), jnp.float32),
            pltpu.SemaphoreType.DMA,
            pltpu.SemaphoreType.DMA,
        ],
        interpret=INTERP,
    )
    def sc_scatter(xs_hbm, ys_hbm, masks_ref,
                   xs_v, ys_v, idx_vs, val_v, sem_in, sem_out):
        wid = lax.axis_index("s")
        base = wid * bpw
        lx = pltpu.async_copy(xs_hbm.at[pl.ds(base, bpw)], xs_v, sem_in)
        ly = pltpu.async_copy(ys_hbm.at[pl.ds(base, bpw)], ys_v, sem_in)
        for j in range(8):
            val_v[pl.ds(j * 16, 16)] = jnp.full((16,), -1.0, jnp.float32)
        lx.wait()
        ly.wait()
        for j in range(bpw // 16):
            xs = xs_v[pl.ds(j * 16, 16)].astype(jnp.int32)
            ys = ys_v[pl.ds(j * 16, 16)].astype(jnp.int32)
            cfg = base + j * 16 + lax.iota(jnp.int32, 16)
            idx_vs[j // 8][pl.ds((j % 8) * 16, 16)] = \
                cfg * (LAT * LAT) + ys * LAT + xs
        stores = [
            pltpu.async_copy(val_v, masks_ref.at[idx_vs[d]], sem_out)
            for d in range(ndma)
        ]
        for c in stores:
            c.wait()

    return sc_scatter


@jax.jit
def _run(x_seps, y_seps):
    n = x_seps.shape[0]
    filled = _tc_fill(n)
    flat = jax.new_ref(filled.reshape(n * LAT * LAT))
    _make_sc_scatter(n)(x_seps, y_seps, flat)
    return jax.freeze(flat).reshape(n, 1, LAT, LAT)


def kernel(x_seps, y_seps):
    return _run(x_seps, y_seps)
